# Initial kernel scaffold; baseline (speedup 1.0000x reference)
import jax
import jax.numpy as jnp
from jax.experimental import pallas as pl


def kernel(x, edge_index, params):
    raise NotImplementedError("write your pallas kernel here")


# speedup vs baseline: 2.0652x; 2.0652x over previous
import functools

import jax
import jax.numpy as jnp
from jax import lax
from jax.experimental import pallas as pl
from jax.experimental.pallas import tpu as pltpu
from jax.experimental.pallas import tpu_sc as plsc

N = 10000
E = 160000
D = 256
EMB = 128
NB_LAYERS = 3
MB = 2000
NBLK = N // MB
NC, NS = 2, 16
K = 128
NCHUNK = E // K


def _mm_body(x_ref, w_ref, b_ref, o_ref, *, relu):
    acc = jnp.dot(x_ref[...], w_ref[...], preferred_element_type=jnp.float32)
    acc = acc + b_ref[...]
    if relu:
        acc = jnp.maximum(acc, 0.0)
    o_ref[...] = acc


def _mm(x, w, b, relu=False):
    m, k = x.shape
    h = w.shape[1]
    mb = min(m, MB)
    return pl.pallas_call(
        functools.partial(_mm_body, relu=relu),
        grid=(m // mb,),
        in_specs=[
            pl.BlockSpec((mb, k), lambda i: (i, 0)),
            pl.BlockSpec((k, h), lambda i: (0, 0)),
            pl.BlockSpec((1, h), lambda i: (0, 0)),
        ],
        out_specs=pl.BlockSpec((mb, h), lambda i: (i, 0)),
        out_shape=jax.ShapeDtypeStruct((m, h), jnp.float32),
    )(x, w, b.reshape(1, h))


def _stats_body(x_ref, o_ref):
    i = pl.program_id(0)

    @pl.when(i == 0)
    def _():
        o_ref[...] = jnp.zeros_like(o_ref)

    xb = x_ref[...]
    o_ref[0:1, :] += jnp.sum(xb, axis=0, keepdims=True)
    o_ref[1:2, :] += jnp.sum(xb * xb, axis=0, keepdims=True)


def _stats(x):
    return pl.pallas_call(
        _stats_body,
        grid=(NBLK,),
        in_specs=[pl.BlockSpec((MB, D), lambda i: (i, 0))],
        out_specs=pl.BlockSpec((2, D), lambda i: (0, 0)),
        out_shape=jax.ShapeDtypeStruct((2, D), jnp.float32),
    )(x)


def _bnpq_body(x_ref, st_ref, g_ref, be_ref, w_ref, b_ref, xbn_ref, pq_ref):
    m = st_ref[0:1, :] * (1.0 / N)
    v = st_ref[1:2, :] * (1.0 / N) - m * m
    inv = lax.rsqrt(v + 1e-5)
    xbn = (x_ref[...] - m) * inv * g_ref[...] + be_ref[...]
    xbn_ref[...] = xbn
    pq_ref[...] = (
        jnp.dot(xbn, w_ref[...], preferred_element_type=jnp.float32) + b_ref[...]
    )


def _bnpq(x, stats, gamma, beta, wcat, bcat):
    return pl.pallas_call(
        _bnpq_body,
        grid=(NBLK,),
        in_specs=[
            pl.BlockSpec((MB, D), lambda i: (i, 0)),
            pl.BlockSpec((2, D), lambda i: (0, 0)),
            pl.BlockSpec((1, D), lambda i: (0, 0)),
            pl.BlockSpec((1, D), lambda i: (0, 0)),
            pl.BlockSpec((D, 2 * D), lambda i: (0, 0)),
            pl.BlockSpec((1, 2 * D), lambda i: (0, 0)),
        ],
        out_specs=[
            pl.BlockSpec((MB, D), lambda i: (i, 0)),
            pl.BlockSpec((MB, 2 * D), lambda i: (i, 0)),
        ],
        out_shape=[
            jax.ShapeDtypeStruct((N, D), jnp.float32),
            jax.ShapeDtypeStruct((N, 2 * D), jnp.float32),
        ],
    )(x, stats, gamma.reshape(1, D), beta.reshape(1, D), wcat, bcat.reshape(1, 2 * D))


def _lin_body(x_ref, xp_ref, wa_ref, wb0_ref, wb1_ref, b_ref, o_ref):
    acc = jnp.dot(x_ref[...], wa_ref[...], preferred_element_type=jnp.float32)
    acc += jnp.dot(xp_ref[0], wb0_ref[...], preferred_element_type=jnp.float32)
    acc += jnp.dot(xp_ref[1], wb1_ref[...], preferred_element_type=jnp.float32)
    o_ref[...] = jnp.maximum(acc + b_ref[...], 0.0)


def _lin(x, xprop2, wa, wb0, wb1, b):
    return pl.pallas_call(
        _lin_body,
        grid=(NBLK,),
        in_specs=[
            pl.BlockSpec((MB, D), lambda i: (i, 0)),
            pl.BlockSpec((2, MB, EMB), lambda i: (0, i, 0)),
            pl.BlockSpec((D, D), lambda i: (0, 0)),
            pl.BlockSpec((EMB, D), lambda i: (0, 0)),
            pl.BlockSpec((EMB, D), lambda i: (0, 0)),
            pl.BlockSpec((1, D), lambda i: (0, 0)),
        ],
        out_specs=pl.BlockSpec((MB, D), lambda i: (i, 0)),
        out_shape=jax.ShapeDtypeStruct((N, D), jnp.float32),
    )(x, xprop2, wa, wb0, wb1, b.reshape(1, D))


def _mlp3_body(x_ref, w0_ref, b0_ref, w1_ref, b1_ref, wf_ref, bf_ref, o_ref):
    h = jnp.maximum(
        jnp.dot(x_ref[...], w0_ref[...], preferred_element_type=jnp.float32)
        + b0_ref[...], 0.0)
    h = jnp.maximum(
        jnp.dot(h, w1_ref[...], preferred_element_type=jnp.float32)
        + b1_ref[...], 0.0)
    o_ref[...] = (
        jnp.dot(h, wf_ref[...], preferred_element_type=jnp.float32) + bf_ref[...]
    )


def _mlp3(x, p):
    m, k = x.shape
    hh = p["W0"].shape[1]
    ho = p["Wf"].shape[1]
    mb = min(m, MB)
    return pl.pallas_call(
        _mlp3_body,
        grid=(m // mb,),
        in_specs=[
            pl.BlockSpec((mb, k), lambda i: (i, 0)),
            pl.BlockSpec((k, hh), lambda i: (0, 0)),
            pl.BlockSpec((1, hh), lambda i: (0, 0)),
            pl.BlockSpec((hh, hh), lambda i: (0, 0)),
            pl.BlockSpec((1, hh), lambda i: (0, 0)),
            pl.BlockSpec((hh, ho), lambda i: (0, 0)),
            pl.BlockSpec((1, ho), lambda i: (0, 0)),
        ],
        out_specs=pl.BlockSpec((mb, ho), lambda i: (i, 0)),
        out_shape=jax.ShapeDtypeStruct((m, ho), jnp.float32),
    )(x, p["W0"], p["b0"].reshape(1, hh), p["W1"], p["b1"].reshape(1, hh),
      p["Wf"], p["bf"].reshape(1, ho))


def _ms_body(x_ref, xa_ref, w0a_ref, w0b_ref, b0_ref, w1_ref, b1_ref,
             wf_ref, bf_ref, o_ref):
    h = jnp.dot(x_ref[...], w0b_ref[...], preferred_element_type=jnp.float32)
    h += jnp.dot(xa_ref[...], w0a_ref[...], preferred_element_type=jnp.float32)
    h = jnp.maximum(h + b0_ref[...], 0.0)
    h = jnp.maximum(
        jnp.dot(h, w1_ref[...], preferred_element_type=jnp.float32)
        + b1_ref[...], 0.0)
    o_ref[...] = (
        jnp.dot(h, wf_ref[...], preferred_element_type=jnp.float32) + bf_ref[...]
    )


def _ms_head(x, xa1, p):
    hh = p["W0"].shape[1]
    w0a, w0b = p["W0"][:EMB], p["W0"][EMB:]
    return pl.pallas_call(
        _ms_body,
        grid=(NBLK,),
        in_specs=[
            pl.BlockSpec((MB, EMB), lambda i: (i, 0)),
            pl.BlockSpec((1, EMB), lambda i: (0, 0)),
            pl.BlockSpec((EMB, hh), lambda i: (0, 0)),
            pl.BlockSpec((EMB, hh), lambda i: (0, 0)),
            pl.BlockSpec((1, hh), lambda i: (0, 0)),
            pl.BlockSpec((hh, hh), lambda i: (0, 0)),
            pl.BlockSpec((1, hh), lambda i: (0, 0)),
            pl.BlockSpec((hh, 1), lambda i: (0, 0)),
            pl.BlockSpec((1, 1), lambda i: (0, 0)),
        ],
        out_specs=pl.BlockSpec((MB, 1), lambda i: (i, 0)),
        out_shape=jax.ShapeDtypeStruct((N, 1), jnp.float32),
    )(x, xa1.reshape(1, EMB), w0a, w0b, p["b0"].reshape(1, hh), p["W1"],
      p["b1"].reshape(1, hh), p["Wf"], p["bf"].reshape(1, 1))


def _rowsum_mean_body(x_ref, rs_ref, mn_ref):
    i = pl.program_id(0)

    @pl.when(i == 0)
    def _():
        mn_ref[...] = jnp.zeros_like(mn_ref)

    rs_ref[...] = jnp.sum(x_ref[...], axis=1, keepdims=True)
    mn_ref[...] += jnp.sum(x_ref[...], axis=0, keepdims=True) * (1.0 / N)


def _rowsum_mean(x):
    k = x.shape[1]
    return pl.pallas_call(
        _rowsum_mean_body,
        grid=(NBLK,),
        in_specs=[pl.BlockSpec((MB, k), lambda i: (i, 0))],
        out_specs=[
            pl.BlockSpec((MB, 1), lambda i: (i, 0)),
            pl.BlockSpec((1, k), lambda i: (0, 0)),
        ],
        out_shape=[
            jax.ShapeDtypeStruct((N, 1), jnp.float32),
            jax.ShapeDtypeStruct((1, k), jnp.float32),
        ],
    )(x)


_MESH = plsc.VectorSubcoreMesh(
    core_axis_name="c", subcore_axis_name="s", num_cores=NC, num_subcores=NS)


def _sc_w_body(pq2_hbm, dst2_hbm, src2_hbm, bv_hbm, bb_hbm, w_hbm,
               idx_d, idx_s, pd, qs, bv, bb, wv, sem1, sem2):
    c = lax.axis_index("c")
    s = lax.axis_index("s")
    wid = s * NC + c
    pltpu.sync_copy(bv_hbm, bv)
    pltpu.sync_copy(bb_hbm, bb)
    nmine = (NCHUNK - wid + (NC * NS - 1)) // (NC * NS)

    def chunk(i, carry):
        base = (wid + i * NC * NS) * K
        pltpu.sync_copy(dst2_hbm.at[pl.ds(base, K)], idx_d)
        pltpu.sync_copy(src2_hbm.at[pl.ds(base, K)], idx_s)
        cp1 = pltpu.async_copy(pq2_hbm.at[idx_d], pd, sem1)
        cp2 = pltpu.async_copy(pq2_hbm.at[idx_s], qs, sem2)
        cp1.wait()
        cp2.wait()

        lanes = lax.broadcasted_iota(jnp.int32, (16,), 0)

        def edge16(j, carry2):
            tvec = jnp.zeros((16,), jnp.float32)
            for e16 in range(16):
                e = j * 16 + e16
                acc = jnp.zeros((16,), jnp.float32)
                for g in range(D // 16):
                    sl = pl.ds(g * 16, 16)
                    h = jnp.maximum(pd[e, sl] + qs[e, sl], 0.0)
                    acc = acc + h * bv[sl]
                for sh in (8, 4, 2, 1):
                    acc = acc + acc.at[lanes ^ sh].get(mode="promise_in_bounds")
                tvec = jnp.where(lanes == e16, acc, tvec)
            wvec = 1.0 / (1.0 + jnp.exp(-(tvec + bb[...])))
            wv[pl.ds(j * 16, 16)] = wvec
            return carry2

        lax.fori_loop(0, K // 16, edge16, 0, unroll=False)
        pltpu.sync_copy(wv, w_hbm.at[pl.ds(base, K)])
        return carry

    lax.fori_loop(0, nmine, chunk, 0, unroll=False)


def _sc_w(pq2, dst2, src2, bvec, bb16):
    kern = pl.kernel(
        _sc_w_body,
        out_type=jax.ShapeDtypeStruct((E,), jnp.float32),
        mesh=_MESH,
        scratch_types=[
            pltpu.VMEM((K,), jnp.int32),
            pltpu.VMEM((K,), jnp.int32),
            pltpu.VMEM((K, D), jnp.float32),
            pltpu.VMEM((K, D), jnp.float32),
            pltpu.VMEM((D,), jnp.float32),
            pltpu.VMEM((16,), jnp.float32),
            pltpu.VMEM((K,), jnp.float32),
            pltpu.SemaphoreType.DMA,
            pltpu.SemaphoreType.DMA,
        ],
    )
    return kern(pq2, dst2, src2, bvec, bb16)


def _sc_scatter_body(xh2_hbm, srcab_hbm, dst_hbm, w_hbm, zero_hbm, xp_hbm,
                     idx_s, idx_d, wv, xr, acc, sem):
    c = lax.axis_index("c")
    s = lax.axis_index("s")
    rows_a = 632
    rows_last = N - 15 * rows_a
    r0 = s * rows_a

    @pl.when(s < NS - 1)
    def _():
        pltpu.sync_copy(zero_hbm.at[pl.ds(r0, rows_a)], acc.at[pl.ds(r0, rows_a)])

    @pl.when(s == NS - 1)
    def _():
        pltpu.sync_copy(zero_hbm.at[pl.ds(r0, rows_last)],
                        acc.at[pl.ds(r0, rows_last)])

    plsc.subcore_barrier()
    nmine = (NCHUNK - s + (NS - 1)) // NS

    def chunk(i, carry):
        base = (s + i * NS) * K
        pltpu.sync_copy(srcab_hbm.at[c, pl.ds(base, K)], idx_s)
        pltpu.sync_copy(dst_hbm.at[pl.ds(base, K)], idx_d)
        pltpu.sync_copy(w_hbm.at[pl.ds(base, K)], wv)
        pltpu.async_copy(xh2_hbm.at[idx_s], xr, sem).wait()

        def edge16(j, carry2):
            wvec = wv[pl.ds(j * 16, 16)]
            for e16 in range(16):
                e = j * 16 + e16
                wf = jnp.full((16,), wvec[e16], jnp.float32)
                for g in range(EMB // 16):
                    sl = pl.ds(g * 16, 16)
                    xr[e, sl] = xr[e, sl] * wf
            return carry2

        lax.fori_loop(0, K // 16, edge16, 0, unroll=False)
        pltpu.sync_copy(xr, acc.at[idx_d], add=True)
        return carry

    lax.fori_loop(0, nmine, chunk, 0, unroll=False)
    plsc.subcore_barrier()

    @pl.when(s < NS - 1)
    def _():
        pltpu.sync_copy(acc.at[pl.ds(r0, rows_a)],
                        xp_hbm.at[c, pl.ds(r0, rows_a)])

    @pl.when(s == NS - 1)
    def _():
        pltpu.sync_copy(acc.at[pl.ds(r0, rows_last)],
                        xp_hbm.at[c, pl.ds(r0, rows_last)])


def _sc_scatter(xh2, srcab, dst, w, zeros_half):
    kern = pl.kernel(
        _sc_scatter_body,
        out_type=jax.ShapeDtypeStruct((NC, N, EMB), jnp.float32),
        mesh=_MESH,
        scratch_types=[
            pltpu.VMEM((K,), jnp.int32),
            pltpu.VMEM((K,), jnp.int32),
            pltpu.VMEM((K,), jnp.float32),
            pltpu.VMEM((K, EMB), jnp.float32),
            pltpu.VMEM_SHARED((N, EMB), jnp.float32),
            pltpu.SemaphoreType.DMA,
        ],
    )
    return kern(xh2, srcab, dst, w, zeros_half)


def kernel(x, edge_index, params):
    src = edge_index[0]
    dst = edge_index[1]
    dst2 = dst * 2
    src2 = src * 2 + 1
    srcab = jnp.stack([src * 2, src * 2 + 1])
    zeros_half = jnp.zeros((N, EMB), jnp.float32)

    mask2d, _ = _rowsum_mean(x)
    mask = mask2d[:, 0]

    emb = x
    for i in range(NB_LAYERS):
        p = params["gnn%d" % i]
        wcat = jnp.concatenate([p["A_W"][:D], p["A_W"][D:]], axis=1)
        bcat = jnp.concatenate([p["A_b"], jnp.zeros((D,), jnp.float32)])
        if i == 0:
            xbn = emb
            pq = _mm(emb, wcat, bcat)
        else:
            st = _stats(emb)
            xbn, pq = _bnpq(emb, st, p["gamma"], p["beta"], wcat, bcat)
        w_edges = _sc_w(
            pq.reshape(2 * N, D), dst2, src2,
            p["B_W"].reshape(D), jnp.full((16,), p["B_b"][0], jnp.float32))
        xprop2 = _sc_scatter(xbn.reshape(2 * N, EMB), srcab, dst, w_edges,
                             zeros_half)
        emb = _lin(xbn, xprop2, p["lin_W"][:D], p["lin_W"][D:D + EMB],
                   p["lin_W"][D + EMB:], p["lin_b"])

    fp_ = params["final"]
    X = _mm(emb, fp_["W"], fp_["b"])

    k1, k2, k3, k4 = jax.random.split(jax.random.key(42), 4)

    lg1 = _mlp3(X, params["mf"])[:, 0]
    f1 = jax.nn.softmax(lg1, axis=0)
    nb_true = mask.sum().astype(jnp.int32) - 9
    tmask = jnp.where(jnp.arange(N) < nb_true, mask, 0.0)
    f1 = f1 * tmask
    a1 = jax.random.categorical(k1, jnp.log(f1))
    p1 = f1[a1]

    xa1 = X[a1]
    lg2 = _ms_head(X, xa1, params["ms"])[:, 0]
    f2 = jax.nn.softmax(lg2, axis=0) * mask
    f2 = f2.at[a1].set(0.0)
    a2 = jax.random.categorical(k2, jnp.log(f2))
    p2 = f2[a2]

    xcat = jnp.concatenate([xa1, X[a2]]).reshape(1, 2 * EMB)
    lg3 = _mlp3(xcat, params["me"])[0]
    f3 = jax.nn.softmax(lg3, axis=0)
    a3 = jax.random.categorical(k3, jnp.log(f3))
    p3 = f3[a3]

    _, xmean = _rowsum_mean(X)
    fl = _mlp3(xmean, params["mt"])[0, 0]
    fp = jax.nn.sigmoid(fl)
    a4 = (jax.random.uniform(k4) < fp).astype(jnp.int32)
    p4 = jnp.where(a4 == 0, 1.0 - fp, fp)

    actions = jnp.stack([
        a1.astype(jnp.int32), a2.astype(jnp.int32), a3.astype(jnp.int32), a4,
    ]).reshape(1, 4)
    probs = jnp.stack([p1, p2, p3, p4])
    return actions, probs
